# Initial kernel scaffold; baseline (speedup 1.0000x reference)
import math

import jax
import jax.numpy as jnp
from jax.experimental import pallas as pl
from jax.experimental.pallas import tpu as pltpu

_N = 16384
_SIZE = 1000
_SMOOTH = 0.1
_CONF = 1.0 - _SMOOTH
_S = _SMOOTH / (_SIZE - 2)
_K = (_SIZE - 2) * _S * math.log(_S) + _CONF * math.log(_CONF)

_ROWS_PER_BLOCK = 1024
_GRID = _N // _ROWS_PER_BLOCK


def _tc_body(x_ref, tgt_ref, out_ref):
    i = pl.program_id(0)

    @pl.when(i == 0)
    def _init():
        out_ref[0, 0] = 0.0

    x = x_ref[...]
    tgt = tgt_ref[...]
    valid = (tgt != 0)
    rowsum = jnp.sum(x, axis=1, keepdims=True)
    x0 = x[:, 0:1]
    cols = jax.lax.broadcasted_iota(jnp.int32, x.shape, 1)
    pick = jnp.sum(jnp.where(cols == tgt, x, 0.0), axis=1, keepdims=True)
    per_row = _K - _S * rowsum + _S * x0 - (_CONF - _S) * pick
    out_ref[0, 0] += jnp.sum(jnp.where(valid, per_row, 0.0))


def kernel(x, target):
    tgt = target.astype(jnp.int32).reshape(_N, 1)
    out = pl.pallas_call(
        _tc_body,
        grid=(_GRID,),
        in_specs=[
            pl.BlockSpec((_ROWS_PER_BLOCK, _SIZE), lambda i: (i, 0)),
            pl.BlockSpec((_ROWS_PER_BLOCK, 1), lambda i: (i, 0)),
        ],
        out_specs=pl.BlockSpec((1, 1), lambda i: (0, 0)),
        out_shape=jax.ShapeDtypeStruct((1, 1), jnp.float32),
        compiler_params=pltpu.CompilerParams(
            dimension_semantics=("arbitrary",),
        ),
    )(x, tgt)
    return out[0, 0]


# speedup vs baseline: 2.3076x; 2.3076x over previous
import math

import jax
import jax.numpy as jnp
from jax.experimental import pallas as pl
from jax.experimental.pallas import tpu as pltpu

_N = 16384
_SIZE = 1000
_SMOOTH = 0.1
_CONF = 1.0 - _SMOOTH
_S = _SMOOTH / (_SIZE - 2)
_K = (_SIZE - 2) * _S * math.log(_S) + _CONF * math.log(_CONF)

_ROWS_PER_BLOCK = 1024
_GRID = _N // _ROWS_PER_BLOCK


def _tc_body(x_ref, tgt_ref, out_ref):
    i = pl.program_id(0)

    @pl.when(i == 0)
    def _init():
        out_ref[...] = jnp.zeros((1, 1), jnp.float32)

    x = x_ref[...]
    tgt = tgt_ref[...]
    valid = (tgt != 0)
    rowsum = jnp.sum(x, axis=1, keepdims=True)
    x0 = x[:, 0:1]
    cols = jax.lax.broadcasted_iota(jnp.int32, x.shape, 1)
    pick = jnp.sum(jnp.where(cols == tgt, x, 0.0), axis=1, keepdims=True)
    per_row = _K - _S * rowsum + _S * x0 - (_CONF - _S) * pick
    out_ref[...] += jnp.sum(jnp.where(valid, per_row, 0.0)).reshape(1, 1)


def kernel(x, target):
    tgt = target.astype(jnp.int32).reshape(_N, 1)
    out = pl.pallas_call(
        _tc_body,
        grid=(_GRID,),
        in_specs=[
            pl.BlockSpec((_ROWS_PER_BLOCK, _SIZE), lambda i: (i, 0)),
            pl.BlockSpec((_ROWS_PER_BLOCK, 1), lambda i: (i, 0)),
        ],
        out_specs=pl.BlockSpec((1, 1), lambda i: (0, 0)),
        out_shape=jax.ShapeDtypeStruct((1, 1), jnp.float32),
        compiler_params=pltpu.CompilerParams(
            dimension_semantics=("arbitrary",),
        ),
    )(x, tgt)
    return out[0, 0]
